# Initial kernel scaffold; baseline (speedup 1.0000x reference)
import jax
import jax.numpy as jnp
from jax.experimental import pallas as pl


def kernel(x, Wr, router_bias, W_up, W_gate, W_down, Ws_up, Ws_gate, Ws_down):
    raise NotImplementedError("write your pallas kernel here")


# speedup vs baseline: 1.4019x; 1.4019x over previous
import functools

import jax
import jax.numpy as jnp
from jax import lax
from jax.experimental import pallas as pl
from jax.experimental.pallas import tpu as pltpu

_E = 8
_K = 2
_TB = 256


def _silu(v):
    return v / (1.0 + jnp.exp(-v))


def _dot_t(a, b):
    return lax.dot_general(
        a.astype(jnp.bfloat16), b.astype(jnp.bfloat16),
        (((1,), (1,)), ((), ())), preferred_element_type=jnp.float32)


def _router_comb(logits, bias):
    lb = logits + bias
    ii = lax.broadcasted_iota(jnp.int32, lb.shape, 1)
    m1 = jnp.max(lb, axis=1, keepdims=True)
    i1 = jnp.min(jnp.where(lb == m1, ii, _E), axis=1, keepdims=True)
    lb2 = jnp.where(ii == i1, -jnp.inf, lb)
    m2 = jnp.max(lb2, axis=1, keepdims=True)
    i2 = jnp.min(jnp.where(lb2 == m2, ii, _E), axis=1, keepdims=True)
    ex = jnp.exp(logits - jnp.max(logits, axis=1, keepdims=True))
    sc = ex / jnp.sum(ex, axis=1, keepdims=True)
    s1 = jnp.sum(jnp.where(ii == i1, sc, 0.0), axis=1, keepdims=True)
    s2 = jnp.sum(jnp.where(ii == i2, sc, 0.0), axis=1, keepdims=True)
    tot = s1 + s2
    return jnp.where(ii == i1, s1 / tot, 0.0) + jnp.where(ii == i2, s2 / tot, 0.0)


def _moe_body(x_ref, lg_ref, bias_ref, wu_ref, wg_ref, wd_ref,
              su_ref, sg_ref, sd_ref, out_ref, acc_ref, comb_ref):
    s = pl.program_id(1)
    x_blk = x_ref[...]

    @pl.when(s == 0)
    def _():
        comb_ref[...] = _router_comb(lg_ref[...], bias_ref[...])
        acc_ref[...] = jnp.zeros_like(acc_ref)

    @pl.when(s < _E)
    def _():
        up = _dot_t(x_blk, wu_ref[0])
        gate = _dot_t(x_blk, wg_ref[0])
        hid = _silu(up) * gate
        eo = lax.dot_general(hid.astype(jnp.bfloat16),
                             wd_ref[0].astype(jnp.bfloat16),
                             (((1,), (1,)), ((), ())),
                             preferred_element_type=jnp.float32)
        comb = comb_ref[...]
        ii = lax.broadcasted_iota(jnp.int32, comb.shape, 1)
        ce = jnp.sum(jnp.where(ii == s, comb, 0.0), axis=1, keepdims=True)
        acc_ref[...] += eo * ce

    @pl.when(s == _E)
    def _():
        up = _dot_t(x_blk, su_ref[0])
        gate = _dot_t(x_blk, sg_ref[0])
        hid = _silu(up) * gate
        eo = lax.dot_general(hid.astype(jnp.bfloat16),
                             sd_ref[0].astype(jnp.bfloat16),
                             (((1,), (1,)), ((), ())),
                             preferred_element_type=jnp.float32)
        out_ref[...] = acc_ref[...] + eo


@jax.jit
def kernel(x, Wr, router_bias, W_up, W_gate, W_down, Ws_up, Ws_gate, Ws_down):
    Bn, Tn, Dn = x.shape
    N = Bn * Tn
    E, H, D = W_up.shape
    HS = Ws_up.shape[1]
    flat = x.reshape(N, D)
    logits = flat @ Wr.T
    nb = N // _TB

    grid = (nb, _E + 1)
    out = pl.pallas_call(
        _moe_body,
        grid=grid,
        in_specs=[
            pl.BlockSpec((_TB, D), lambda tb, s: (tb, 0)),
            pl.BlockSpec((_TB, E), lambda tb, s: (tb, 0)),
            pl.BlockSpec((1, E), lambda tb, s: (0, 0)),
            pl.BlockSpec((1, H, D), lambda tb, s: (jnp.minimum(s, _E - 1), 0, 0)),
            pl.BlockSpec((1, H, D), lambda tb, s: (jnp.minimum(s, _E - 1), 0, 0)),
            pl.BlockSpec((1, D, H), lambda tb, s: (jnp.minimum(s, _E - 1), 0, 0)),
            pl.BlockSpec((1, HS, D), lambda tb, s: (0, 0, 0)),
            pl.BlockSpec((1, HS, D), lambda tb, s: (0, 0, 0)),
            pl.BlockSpec((1, D, HS), lambda tb, s: (0, 0, 0)),
        ],
        out_specs=pl.BlockSpec((_TB, D), lambda tb, s: (tb, 0)),
        out_shape=jax.ShapeDtypeStruct((N, D), jnp.float32),
        scratch_shapes=[
            pltpu.VMEM((_TB, D), jnp.float32),
            pltpu.VMEM((_TB, E), jnp.float32),
        ],
        compiler_params=pltpu.CompilerParams(
            dimension_semantics=("arbitrary", "arbitrary")),
    )(flat, logits, router_bias.reshape(1, E), W_up, W_gate, W_down,
      Ws_up, Ws_gate, Ws_down)
    return out.reshape(Bn, Tn, Dn)


# speedup vs baseline: 1.5987x; 1.1404x over previous
import jax
import jax.numpy as jnp
from jax import lax
from jax.experimental import pallas as pl
from jax.experimental.pallas import tpu as pltpu

_E = 8
_TB = 256


def _silu(v):
    return v / (1.0 + jnp.exp(-v))


def _dot_t(a, b):
    return lax.dot_general(a, b, (((1,), (1,)), ((), ())),
                           preferred_element_type=jnp.float32)


def _router_comb(logits, bias):
    lb = logits + bias
    ii = lax.broadcasted_iota(jnp.int32, lb.shape, 1)
    m1 = jnp.max(lb, axis=1, keepdims=True)
    i1 = jnp.min(jnp.where(lb == m1, ii, _E), axis=1, keepdims=True)
    lb2 = jnp.where(ii == i1, -jnp.inf, lb)
    m2 = jnp.max(lb2, axis=1, keepdims=True)
    i2 = jnp.min(jnp.where(lb2 == m2, ii, _E), axis=1, keepdims=True)
    ex = jnp.exp(logits - jnp.max(logits, axis=1, keepdims=True))
    sc = ex / jnp.sum(ex, axis=1, keepdims=True)
    s1 = jnp.sum(jnp.where(ii == i1, sc, 0.0), axis=1, keepdims=True)
    s2 = jnp.sum(jnp.where(ii == i2, sc, 0.0), axis=1, keepdims=True)
    tot = s1 + s2
    return jnp.where(ii == i1, s1 / tot, 0.0) + jnp.where(ii == i2, s2 / tot, 0.0)


def _moe_body(x_ref, lg_ref, bias_ref, wu_ref, wg_ref, wd_ref, out_ref):
    ns = wu_ref.shape[0]
    x_blk = x_ref[...]
    comb = _router_comb(lg_ref[...], bias_ref[...])
    acc = None
    for s in range(ns):
        up = _dot_t(x_blk, wu_ref[s])
        gate = _dot_t(x_blk, wg_ref[s])
        hid = _silu(up) * gate
        eo = lax.dot_general(hid.astype(jnp.bfloat16), wd_ref[s],
                             (((1,), (0,)), ((), ())),
                             preferred_element_type=jnp.float32)
        if s < _E:
            eo = eo * comb[:, s:s + 1]
        acc = eo if acc is None else acc + eo
    out_ref[...] = acc


@jax.jit
def kernel(x, Wr, router_bias, W_up, W_gate, W_down, Ws_up, Ws_gate, Ws_down):
    Bn, Tn, Dn = x.shape
    N = Bn * Tn
    E, H, D = W_up.shape
    HS = Ws_up.shape[1]
    NSH = HS // H
    flat = x.reshape(N, D)
    logits = flat @ Wr.T
    xbf = flat.astype(jnp.bfloat16)
    bf = jnp.bfloat16

    up_all = jnp.concatenate(
        [W_up.astype(bf), Ws_up[0].astype(bf).reshape(NSH, H, D)], axis=0)
    gate_all = jnp.concatenate(
        [W_gate.astype(bf), Ws_gate[0].astype(bf).reshape(NSH, H, D)], axis=0)
    down_all = jnp.concatenate(
        [W_down.astype(bf).transpose(0, 2, 1),
         Ws_down[0].astype(bf).T.reshape(NSH, H, D)], axis=0)

    nb = N // _TB
    ns = E + NSH
    out = pl.pallas_call(
        _moe_body,
        grid=(nb,),
        in_specs=[
            pl.BlockSpec((_TB, D), lambda tb: (tb, 0)),
            pl.BlockSpec((_TB, E), lambda tb: (tb, 0)),
            pl.BlockSpec((1, E), lambda tb: (0, 0)),
            pl.BlockSpec((ns, H, D), lambda tb: (0, 0, 0)),
            pl.BlockSpec((ns, H, D), lambda tb: (0, 0, 0)),
            pl.BlockSpec((ns, H, D), lambda tb: (0, 0, 0)),
        ],
        out_specs=pl.BlockSpec((_TB, D), lambda tb: (tb, 0)),
        out_shape=jax.ShapeDtypeStruct((N, D), jnp.float32),
        compiler_params=pltpu.CompilerParams(
            dimension_semantics=("arbitrary",)),
    )(xbf, logits, router_bias.reshape(1, E), up_all, gate_all, down_all)
    return out.reshape(Bn, Tn, Dn)
